# Initial kernel scaffold; baseline (speedup 1.0000x reference)
import jax
import jax.numpy as jnp
from jax.experimental import pallas as pl


def kernel(variable_features, row_features, edge_index, edge_values, params):
    raise NotImplementedError("write your pallas kernel here")


# speedup vs baseline: 2.8769x; 2.8769x over previous
import functools

import jax
import jax.numpy as jnp
from jax import lax
from jax.experimental import pallas as pl
from jax.experimental.pallas import tpu as pltpu
from jax.experimental.pallas import tpu_sc as plsc

NV = 50000
NCN = 50000
NE = 800000
EMB = 64
VAR_NF = 43
CONS_NF = 5

HALF = 25000
ACC = 25088
RPT = ACC // 16
CH = 1024
SUB = CH // 128
NEP = 802816
NCH = NEP // CH
NW = 32


@functools.lru_cache(maxsize=None)
def _build_sc_gather():
  mesh = plsc.VectorSubcoreMesh(core_axis_name="c", subcore_axis_name="s")

  @functools.partial(
      pl.kernel,
      out_type=(
          jax.ShapeDtypeStruct((NEP, EMB), jnp.float32),
          jax.ShapeDtypeStruct((NEP, EMB), jnp.float32),
      ),
      mesh=mesh,
      scratch_types=[
          pltpu.VMEM((SUB, 128), jnp.int32),
          pltpu.VMEM((SUB, 128), jnp.int32),
          pltpu.VMEM((CH, EMB), jnp.float32),
          pltpu.SemaphoreType.DMA,
      ],
      compiler_params=pltpu.CompilerParams(use_tc_tiling_on_sc=False),
  )
  def _sc_gather_k(lt_hbm, rt_hbm, si_hbm, di_hbm, oa_hbm, ob_hbm,
                   si_v, di_v, buf_a, sem):
    c = lax.axis_index("c")
    s = lax.axis_index("s")
    wid = s * 2 + c
    base = NCH // NW
    nk = base + jnp.where(wid < (NCH - base * NW), 1, 0)

    def chunk(k, carry):
      cid = wid + k * NW
      pltpu.sync_copy(si_hbm.at[pl.ds(cid * SUB, SUB)], si_v)
      pltpu.sync_copy(di_hbm.at[pl.ds(cid * SUB, SUB)], di_v)
      descs = [pltpu.async_copy(
          lt_hbm.at[si_v.at[j]], buf_a.at[pl.ds(j * 128, 128)], sem)
          for j in range(SUB)]
      for d in descs:
        d.wait()
      pltpu.sync_copy(buf_a, oa_hbm.at[pl.ds(cid * CH, CH)])
      descs = [pltpu.async_copy(
          rt_hbm.at[di_v.at[j]], buf_a.at[pl.ds(j * 128, 128)], sem)
          for j in range(SUB)]
      for d in descs:
        d.wait()
      pltpu.sync_copy(buf_a, ob_hbm.at[pl.ds(cid * CH, CH)])
      return carry

    lax.fori_loop(0, nk, chunk, 0)

  return _sc_gather_k


def _sc_gather(lt, rt, si2, di2):
  return _build_sc_gather()(lt, rt, si2, di2)


def _local_idx(di_v, li_v, lo, s):
  for j in range(SUB):
    for u in range(8):
      v = di_v[j, pl.ds(u * 16, 16)]
      li = v - lo
      ok = (li >= 0) & (li < HALF)
      li_v[j, pl.ds(u * 16, 16)] = jnp.where(ok, li, HALF + s)


@functools.lru_cache(maxsize=None)
def _build_sc_scatter():
  mesh = plsc.VectorSubcoreMesh(core_axis_name="c", subcore_axis_name="s")

  @functools.partial(
      pl.kernel,
      out_type=jax.ShapeDtypeStruct((2, ACC, EMB), jnp.float32),
      mesh=mesh,
      scratch_types=[
          pltpu.VMEM((SUB, 128), jnp.int32),
          pltpu.VMEM((SUB, 128), jnp.int32),
          pltpu.VMEM((256, EMB), jnp.float32),
          pltpu.VMEM_SHARED((ACC, EMB), jnp.float32),
      ],
      compiler_params=pltpu.CompilerParams(use_tc_tiling_on_sc=False),
  )
  def _sc_scatter_k(x_hbm, di_hbm, z64_hbm, s_out, di_v, li_v, x_v, acc):
    c = lax.axis_index("c")
    s = lax.axis_index("s")

    @pl.when(s == 0)
    def _():
      pltpu.sync_copy(z64_hbm, acc)

    plsc.subcore_barrier()
    lo = c * HALF

    def chunk(k, carry):
      cid = s + k * 16
      pltpu.sync_copy(di_hbm.at[pl.ds(cid * SUB, SUB)], di_v)
      _local_idx(di_v, li_v, lo, s)
      for q in range(4):
        pltpu.sync_copy(x_hbm.at[pl.ds(cid * CH + q * 256, 256)], x_v)
        pltpu.sync_copy(x_v.at[pl.ds(0, 128)],
                        acc.at[li_v.at[2 * q]], add=True)
        pltpu.sync_copy(x_v.at[pl.ds(128, 128)],
                        acc.at[li_v.at[2 * q + 1]], add=True)
      return carry

    lax.fori_loop(0, NCH // 16, chunk, 0)
    plsc.subcore_barrier()
    pltpu.sync_copy(acc.at[pl.ds(s * RPT, RPT)],
                    s_out.at[c, pl.ds(s * RPT, RPT)])

  return _sc_scatter_k


def _sc_scatter(x, di2, z64):
  return _build_sc_scatter()(x, di2, z64)


@functools.lru_cache(maxsize=None)
def _build_sc_counts():
  mesh = plsc.VectorSubcoreMesh(core_axis_name="c", subcore_axis_name="s")

  @functools.partial(
      pl.kernel,
      out_type=(
          jax.ShapeDtypeStruct((2, ACC, 16), jnp.float32),
          jax.ShapeDtypeStruct((2, ACC, 16), jnp.float32),
      ),
      mesh=mesh,
      scratch_types=[
          pltpu.VMEM((SUB, 128), jnp.int32),
          pltpu.VMEM((SUB, 128), jnp.int32),
          pltpu.VMEM((128, 16), jnp.float32),
          pltpu.VMEM_SHARED((ACC, 16), jnp.float32),
          pltpu.VMEM_SHARED((ACC, 16), jnp.float32),
      ],
      compiler_params=pltpu.CompilerParams(use_tc_tiling_on_sc=False),
  )
  def _sc_counts_k(ci_hbm, vi_hbm, z16_hbm, ones_hbm, cc_out, cv_out,
                   di_v, li_v, ones_v, cacc_c, cacc_v):
    c = lax.axis_index("c")
    s = lax.axis_index("s")
    pltpu.sync_copy(ones_hbm, ones_v)

    @pl.when(s == 0)
    def _():
      pltpu.sync_copy(z16_hbm, cacc_c)
      pltpu.sync_copy(z16_hbm, cacc_v)

    plsc.subcore_barrier()
    lo = c * HALF

    def chunk(k, carry):
      cid = s + k * 16
      pltpu.sync_copy(ci_hbm.at[pl.ds(cid * SUB, SUB)], di_v)
      _local_idx(di_v, li_v, lo, s)
      for j in range(SUB):
        pltpu.sync_copy(ones_v, cacc_c.at[li_v.at[j]], add=True)
      pltpu.sync_copy(vi_hbm.at[pl.ds(cid * SUB, SUB)], di_v)
      _local_idx(di_v, li_v, lo, s)
      for j in range(SUB):
        pltpu.sync_copy(ones_v, cacc_v.at[li_v.at[j]], add=True)
      return carry

    lax.fori_loop(0, NCH // 16, chunk, 0)
    plsc.subcore_barrier()
    pltpu.sync_copy(cacc_c.at[pl.ds(s * RPT, RPT)],
                    cc_out.at[c, pl.ds(s * RPT, RPT)])
    pltpu.sync_copy(cacc_v.at[pl.ds(s * RPT, RPT)],
                    cv_out.at[c, pl.ds(s * RPT, RPT)])

  return _sc_counts_k


def _sc_counts(ci2, vi2, z16, ones):
  return _build_sc_counts()(ci2, vi2, z16, ones)


def _mm(x, w):
    return lax.dot_general(x, w, (((1,), (1,)), ((), ())),
                           preferred_element_type=jnp.float32)


def _lk(x):
    return jnp.where(x >= 0, x, 0.01 * x)


def _ln_tc(x, g, b):
    m = jnp.mean(x, axis=-1, keepdims=True)
    xc = x - m
    v = jnp.mean(xc * xc, axis=-1, keepdims=True)
    return xc * lax.rsqrt(v + 1e-5) * g + b


def _full_spec(a):
    nd = a.ndim
    return pl.BlockSpec(a.shape, lambda i, _n=nd: (0,) * _n)


def _row_spec(nb, d):
    return pl.BlockSpec((nb, d), lambda i: (i, 0))


def _embed(x, nf, lng, lnb, w1, b1, w2, b2, lw, lb, rw):
    nb = 5000

    def body(x_r, lng_r, lnb_r, w1_r, b1_r, w2_r, b2_r, lw_r, lb_r, rw_r,
             oe, ol, orr):
        xn = _ln_tc(x_r[...], lng_r[...], lnb_r[...])
        h = _lk(_mm(xn, w1_r[...]) + b1_r[...])
        e = _lk(_mm(h, w2_r[...]) + b2_r[...])
        oe[...] = e
        ol[...] = _mm(e, lw_r[...]) + lb_r[...]
        orr[...] = _mm(e, rw_r[...])

    n = x.shape[0]
    params = (lng, lnb, w1, b1, w2, b2, lw, lb, rw)
    return pl.pallas_call(
        body,
        grid=(n // nb,),
        in_specs=[_row_spec(nb, nf)] + [_full_spec(a) for a in params],
        out_specs=[_row_spec(nb, EMB)] * 3,
        out_shape=[jax.ShapeDtypeStruct((n, EMB), jnp.float32)] * 3,
    )(x, *params)


def _add_ln_leaky(ga, gb, g, b, fin_w):
    nb = 8192

    def body(a_r, b_r, g_r, bb_r, w_r, o):
        t = _lk(_ln_tc(a_r[...] + b_r[...], g_r[...], bb_r[...]))
        o[...] = _mm(t, w_r[...])

    return pl.pallas_call(
        body,
        grid=(NEP // nb,),
        in_specs=[_row_spec(nb, EMB), _row_spec(nb, EMB),
                  _full_spec(g), _full_spec(b), _full_spec(fin_w)],
        out_specs=_row_spec(nb, EMB),
        out_shape=jax.ShapeDtypeStruct((NEP, EMB), jnp.float32),
    )(ga, gb, g, b, fin_w)


def _post(S, cnt, right, fin_b, pg, pb, w1a, w1b, b1, w2, b2,
          tail_w1, tail_b1, tail_w2, tail_b2, head):
    nb = 5000
    dout = 1 if head else EMB

    def body(s_r, cnt_r, r_r, finb_r, pg_r, pb_r, w1a_r, w1b_r, b1_r,
             w2_r, b2_r, tw1_r, tb1_r, tw2_r, tb2_r, o):
        cntv = cnt_r[...]
        sm = s_r[...] + cntv * finb_r[...]
        agg = sm / jnp.maximum(cntv, 1.0)
        h = _ln_tc(agg, pg_r[...], pb_r[...])
        t = _lk(_mm(h, w1a_r[...]) + _mm(r_r[...], w1b_r[...]) + b1_r[...])
        node = _mm(t, w2_r[...]) + b2_r[...]
        if head:
            h2 = _lk(_mm(node, tw1_r[...]) + tb1_r[...])
            o[...] = jnp.sum(h2 * tw2_r[...], axis=-1, keepdims=True) \
                + tb2_r[...]
        else:
            o[...] = _mm(node, tw1_r[...]) + tb1_r[...]

    n = S.shape[0]
    params = (fin_b, pg, pb, w1a, w1b, b1, w2, b2,
              tail_w1, tail_b1, tail_w2, tail_b2)
    return pl.pallas_call(
        body,
        grid=(n // nb,),
        in_specs=[_row_spec(nb, EMB), _row_spec(nb, 1), _row_spec(nb, EMB)]
                 + [_full_spec(a) for a in params],
        out_specs=_row_spec(nb, dout),
        out_shape=jax.ShapeDtypeStruct((n, dout), jnp.float32),
    )(S, cnt, right, *params)


def kernel(variable_features, row_features, edge_index, edge_values, params):
    p = params
    r = lambda v: v.reshape(1, -1)
    npad = NEP - NE
    pad0 = lambda a: jnp.concatenate(
        [a, jnp.zeros((npad,), jnp.int32)]).reshape(NEP // 128, 128)
    padt = lambda a: jnp.concatenate(
        [a, jnp.full((npad,), NCN, jnp.int32)]).reshape(NEP // 128, 128)
    ci_g = pad0(edge_index[0])
    vi_g = pad0(edge_index[1])
    ci_s = padt(edge_index[0])
    vi_s = padt(edge_index[1])
    z64 = jnp.zeros((ACC, EMB), jnp.float32)
    z16 = jnp.zeros((ACC, 16), jnp.float32)
    ones = jnp.ones((128, 16), jnp.float32)

    Cc, Cv = _sc_counts(ci_s, vi_s, z16, ones)
    cnt_c = jnp.concatenate([Cc[0, :HALF, :1], Cc[1, :HALF, :1]], 0)
    cnt_v = jnp.concatenate([Cv[0, :HALF, :1], Cv[1, :HALF, :1]], 0)

    var_emb, lt_vc, rt_cv = _embed(
        variable_features, VAR_NF, r(p['ve_ln_g']), r(p['ve_ln_b']),
        p['ve_w1'], r(p['ve_b1']), p['ve_w2'], r(p['ve_b2']),
        p['vc_left_w'], r(p['vc_left_b']), p['cv_right_w'])
    cons_emb, _unused, rt_vc = _embed(
        row_features, CONS_NF, r(p['ce_ln_g']), r(p['ce_ln_b']),
        p['ce_w1'], r(p['ce_b1']), p['ce_w2'], r(p['ce_b2']),
        p['vc_right_w'], r(jnp.zeros((EMB,), jnp.float32)), p['vc_right_w'])

    ga1, gb1 = _sc_gather(lt_vc, rt_vc, vi_g, ci_g)
    x1 = _add_ln_leaky(ga1, gb1, r(p['vc_fin_ln_g']), r(p['vc_fin_ln_b']),
                       p['vc_fin_w'])
    S1 = _sc_scatter(x1, ci_s, z64)
    S1f = jnp.concatenate([S1[0, :HALF], S1[1, :HALF]], 0)
    w1 = p['vc_out_w1']
    lt_cv = _post(S1f, cnt_c, cons_emb, r(p['vc_fin_b']),
                  r(p['vc_post_ln_g']), r(p['vc_post_ln_b']),
                  w1[:, :EMB], w1[:, EMB:], r(p['vc_out_b1']),
                  p['vc_out_w2'], r(p['vc_out_b2']),
                  p['cv_left_w'], r(p['cv_left_b']),
                  p['cv_left_w'], r(p['cv_left_b']), head=False)

    ga2, gb2 = _sc_gather(lt_cv, rt_cv, ci_g, vi_g)
    x2 = _add_ln_leaky(ga2, gb2, r(p['cv_fin_ln_g']), r(p['cv_fin_ln_b']),
                       p['cv_fin_w'])
    S2 = _sc_scatter(x2, vi_s, z64)
    S2f = jnp.concatenate([S2[0, :HALF], S2[1, :HALF]], 0)
    w1 = p['cv_out_w1']
    out2 = _post(S2f, cnt_v, var_emb, r(p['cv_fin_b']),
                 r(p['cv_post_ln_g']), r(p['cv_post_ln_b']),
                 w1[:, :EMB], w1[:, EMB:], r(p['cv_out_b1']),
                 p['cv_out_w2'], r(p['cv_out_b2']),
                 p['h_w1'], r(p['h_b1']), p['h_w2'], r(p['h_b2']), head=True)
    return out2[:, 0]
